# baseline (speedup 1.0000x reference)
import functools

import jax
import jax.numpy as jnp
from jax import lax
from jax.experimental import pallas as pl
from jax.experimental.pallas import tpu as pltpu
from jax.experimental.pallas import tpu_sc as plsc

N_NODES = 10000
N_EDGES = 320000
D_FEAT = 128
N_REL = 16

NC = 2
NS = 16
NW = NC * NS
LANES = 16

CHUNK = 128
CPW = 80
N_CHUNKS_P = CPW * NW
E_PAD = N_CHUNKS_P * CHUNK
STEPS = CPW // 2
ACC_ROWS = 10240
ROWS_PER_SUB = ACC_ROWS // NS
N_STAGE = ROWS_PER_SUB // CHUNK


def _sc_body(ent_hbm, rel_hbm, meta_hbm, scor_hbm, out_hbm,
             rel_v, meta_v, scor_v, heads_v, tails_v, rows_v, prod_v,
             acc_sh,
             sem_m0, sem_m1, sem_g0, sem_g1, sem_s0, sem_s1):
    cid = lax.axis_index("c")
    sid = lax.axis_index("s")
    wid = sid * NC + cid
    sem_m = (sem_m0, sem_m1)
    sem_g = (sem_g0, sem_g1)
    sem_s = (sem_s0, sem_s1)

    pltpu.sync_copy(rel_hbm, rel_v)

    def _zero_row(i, carry):
        for j in range(D_FEAT // LANES):
            prod_v[i, pl.ds(j * LANES, LANES)] = jnp.zeros((LANES,),
                                                           jnp.float32)
        return carry

    lax.fori_loop(0, CHUNK, _zero_row, 0)
    for k in range(N_STAGE):
        pltpu.sync_copy(
            prod_v,
            acc_sh.at[pl.ds(sid * ROWS_PER_SUB + k * CHUNK, CHUNK)])
    plsc.subcore_barrier()

    def _compute(b):
        @plsc.parallel_loop(0, CHUNK // LANES)
        def _group(g):
            gsl = pl.ds(g * LANES, LANES)
            s16 = scor_v[b, gsl]
            r16 = meta_v[b, 2, gsl]
            for k in range(LANES):
                e = g * LANES + k
                s = s16[k]
                r = r16[k]
                rel_row = [rel_v[r, pl.ds(j * LANES, LANES)]
                           for j in range(D_FEAT // LANES)]
                ent_row = [rows_v[e, pl.ds(j * LANES, LANES)]
                           for j in range(D_FEAT // LANES)]
                for j in range(D_FEAT // LANES):
                    prod_v[e, pl.ds(j * LANES, LANES)] = (
                        ent_row[j] * (rel_row[j] * s))

    c0 = wid

    def _issue_meta(c, m, sem):
        pltpu.async_copy(meta_hbm.at[c], meta_v.at[m], sem)
        pltpu.async_copy(scor_hbm.at[c], scor_v.at[m], sem)

    def _wait_meta(c, m, sem):
        pltpu.make_async_copy(meta_hbm.at[c], meta_v.at[m], sem).wait()
        pltpu.make_async_copy(scor_hbm.at[c], scor_v.at[m], sem).wait()

    _issue_meta(c0, 0, sem_m[0])
    _issue_meta(c0 + NW, 1, sem_m[1])
    _wait_meta(c0, 0, sem_m[0])
    _wait_meta(c0 + NW, 1, sem_m[1])

    def _step(step, carry):
        for b in range(2):
            i = step * 2 + b
            c = wid + i * NW

            @pl.when(step > 0)
            def _():
                _wait_meta(c, b, sem_m[b])

            for g in range(CHUNK // LANES):
                gsl = pl.ds(g * LANES, LANES)
                tails_v[gsl] = meta_v[b, 1, gsl]
                heads_v[gsl] = meta_v[b, 0, gsl]

            pltpu.async_copy(ent_hbm.at[tails_v], rows_v, sem_g[b])
            pltpu.make_async_copy(
                ent_hbm.at[tails_v], rows_v, sem_g[b]).wait()

            _compute(b)

            @pl.when(i + 2 < CPW)
            def _():
                _issue_meta(c + 2 * NW, b, sem_m[b])

            pltpu.async_copy(prod_v, acc_sh.at[heads_v],
                             sem_s[b], add=True)
            pltpu.make_async_copy(prod_v, acc_sh.at[heads_v],
                                  sem_s[b]).wait()
        return carry

    lax.fori_loop(0, STEPS, _step, 0)
    plsc.subcore_barrier()

    for k in range(N_STAGE):
        row0 = sid * ROWS_PER_SUB + k * CHUNK
        pltpu.sync_copy(acc_sh.at[pl.ds(row0, CHUNK)], prod_v)
        pltpu.sync_copy(prod_v, out_hbm.at[cid, pl.ds(row0, CHUNK)])


@functools.cache
def _get_sc_agg():
    return pl.kernel(
        _sc_body,
        out_type=jax.ShapeDtypeStruct((NC, ACC_ROWS, D_FEAT), jnp.float32),
        mesh=plsc.VectorSubcoreMesh(core_axis_name="c", subcore_axis_name="s",
                                    num_cores=NC, num_subcores=NS),
        scratch_types=[
            pltpu.VMEM((N_REL, D_FEAT), jnp.float32),
            pltpu.VMEM((2, 3, CHUNK), jnp.int32),
            pltpu.VMEM((2, CHUNK), jnp.float32),
            pltpu.VMEM((CHUNK,), jnp.int32),
            pltpu.VMEM((CHUNK,), jnp.int32),
            pltpu.VMEM((CHUNK, D_FEAT), jnp.float32),
            pltpu.VMEM((CHUNK, D_FEAT), jnp.float32),
            pltpu.VMEM_SHARED((ACC_ROWS, D_FEAT), jnp.float32),
            pltpu.SemaphoreType.DMA,
            pltpu.SemaphoreType.DMA,
            pltpu.SemaphoreType.DMA,
            pltpu.SemaphoreType.DMA,
            pltpu.SemaphoreType.DMA,
            pltpu.SemaphoreType.DMA,
        ],
    )


def _tc_add_body(parts_ref, out_ref):
    out_ref[...] = parts_ref[0] + parts_ref[1]


def _tc_add(parts):
    rows = 2000
    return pl.pallas_call(
        _tc_add_body,
        out_shape=jax.ShapeDtypeStruct((N_NODES, D_FEAT), jnp.float32),
        grid=(N_NODES // rows,),
        in_specs=[pl.BlockSpec((NC, rows, D_FEAT), lambda i: (0, i, 0))],
        out_specs=pl.BlockSpec((rows, D_FEAT), lambda i: (i, 0)),
    )(parts)


@jax.jit
def kernel(entity_emb, relation_emb, scores, edge_index, edge_type):
    head = edge_index[0].astype(jnp.int32)
    tail = edge_index[1].astype(jnp.int32)
    rel_idx = jnp.remainder(edge_type.astype(jnp.int32) - 1, N_REL)
    pad = E_PAD - N_EDGES
    pad_head = N_NODES + jnp.arange(pad, dtype=jnp.int32) % (ACC_ROWS - N_NODES)
    meta = jnp.stack([
        jnp.concatenate([head, pad_head]),
        jnp.concatenate([tail, jnp.zeros((pad,), jnp.int32)]),
        jnp.concatenate([rel_idx, jnp.zeros((pad,), jnp.int32)]),
    ])
    meta = meta.reshape(3, N_CHUNKS_P, CHUNK).transpose(1, 0, 2)
    scor = jnp.concatenate([scores, jnp.zeros((pad,), jnp.float32)])
    scor = scor.reshape(N_CHUNKS_P, CHUNK)
    parts = _get_sc_agg()(entity_emb, relation_emb, meta, scor)
    return _tc_add(parts)


# speedup vs baseline: 1.7246x; 1.7246x over previous
import functools

import jax
import jax.numpy as jnp
from jax import lax
from jax.experimental import pallas as pl
from jax.experimental.pallas import tpu as pltpu
from jax.experimental.pallas import tpu_sc as plsc

N_NODES = 10000
N_EDGES = 320000
D_FEAT = 128
N_REL = 16

NC = 2
NS = 16
NW = NC * NS
LANES = 16

CHUNK = 128
N_CHUNKS = N_EDGES // CHUNK
ACC_ROWS = 10240
ROWS_PER_SUB = ACC_ROWS // NS
STAGE_ROWS = 128
N_STAGE = ROWS_PER_SUB // STAGE_ROWS


def _sc_body(ent_hbm, rel_hbm, scores_hbm, head_hbm, tail_hbm, relidx_hbm,
             out_hbm,
             rel_v, headi_v, taili_v, relidx_v, scores_v, rows_v, prod_v,
             acc_sh, sem):
    cid = lax.axis_index("c")
    sid = lax.axis_index("s")
    wid = sid * NC + cid

    pltpu.sync_copy(rel_hbm, rel_v)

    def _zero_row(i, carry):
        for j in range(D_FEAT // LANES):
            prod_v[i, pl.ds(j * LANES, LANES)] = jnp.zeros((LANES,),
                                                           jnp.float32)
        return carry

    lax.fori_loop(0, STAGE_ROWS, _zero_row, 0)
    for k in range(N_STAGE):
        pltpu.sync_copy(
            prod_v,
            acc_sh.at[pl.ds(sid * ROWS_PER_SUB + k * STAGE_ROWS, STAGE_ROWS)])
    plsc.subcore_barrier()

    def _chunk(i, carry):
        base = (wid + i * NW) * CHUNK
        pltpu.async_copy(head_hbm.at[pl.ds(base, CHUNK)], headi_v, sem)
        pltpu.async_copy(tail_hbm.at[pl.ds(base, CHUNK)], taili_v, sem)
        pltpu.async_copy(relidx_hbm.at[pl.ds(base, CHUNK)], relidx_v, sem)
        pltpu.async_copy(scores_hbm.at[pl.ds(base, CHUNK)], scores_v, sem)
        pltpu.make_async_copy(head_hbm.at[pl.ds(base, CHUNK)], headi_v,
                              sem).wait()
        pltpu.make_async_copy(tail_hbm.at[pl.ds(base, CHUNK)], taili_v,
                              sem).wait()
        pltpu.make_async_copy(relidx_hbm.at[pl.ds(base, CHUNK)], relidx_v,
                              sem).wait()
        pltpu.make_async_copy(scores_hbm.at[pl.ds(base, CHUNK)], scores_v,
                              sem).wait()
        pltpu.async_copy(ent_hbm.at[taili_v], rows_v, sem).wait()

        def _group(g, c2):
            s16 = scores_v[pl.ds(g * LANES, LANES)]
            r16 = relidx_v[pl.ds(g * LANES, LANES)]
            for k in range(LANES):
                e = g * LANES + k
                s = s16[k]
                r = r16[k]
                for j in range(D_FEAT // LANES):
                    sl = pl.ds(j * LANES, LANES)
                    prod_v[e, sl] = rows_v[e, sl] * (rel_v[r, sl] * s)
            return c2

        @plsc.parallel_loop(0, CHUNK // LANES)
        def _group(g):
            s16 = scores_v[pl.ds(g * LANES, LANES)]
            r16 = relidx_v[pl.ds(g * LANES, LANES)]
            for k in range(LANES):
                e = g * LANES + k
                s = s16[k]
                r = r16[k]
                rel_row = [rel_v[r, pl.ds(j * LANES, LANES)]
                           for j in range(D_FEAT // LANES)]
                ent_row = [rows_v[e, pl.ds(j * LANES, LANES)]
                           for j in range(D_FEAT // LANES)]
                for j in range(D_FEAT // LANES):
                    prod_v[e, pl.ds(j * LANES, LANES)] = (
                        ent_row[j] * (rel_row[j] * s))

        pltpu.sync_copy(prod_v, acc_sh.at[headi_v], add=True)
        return carry

    n_my = (N_CHUNKS - wid + NW - 1) // NW
    lax.fori_loop(0, n_my, _chunk, 0)
    plsc.subcore_barrier()

    for k in range(N_STAGE):
        row0 = sid * ROWS_PER_SUB + k * STAGE_ROWS
        pltpu.sync_copy(acc_sh.at[pl.ds(row0, STAGE_ROWS)], prod_v)
        pltpu.sync_copy(prod_v, out_hbm.at[cid, pl.ds(row0, STAGE_ROWS)])


@functools.cache
def _get_sc_agg():
    return pl.kernel(
        _sc_body,
        out_type=jax.ShapeDtypeStruct((NC, ACC_ROWS, D_FEAT), jnp.float32),
        mesh=plsc.VectorSubcoreMesh(core_axis_name="c", subcore_axis_name="s",
                                    num_cores=NC, num_subcores=NS),
        scratch_types=[
            pltpu.VMEM((N_REL, D_FEAT), jnp.float32),
            pltpu.VMEM((CHUNK,), jnp.int32),
            pltpu.VMEM((CHUNK,), jnp.int32),
            pltpu.VMEM((CHUNK,), jnp.int32),
            pltpu.VMEM((CHUNK,), jnp.float32),
            pltpu.VMEM((CHUNK, D_FEAT), jnp.float32),
            pltpu.VMEM((CHUNK, D_FEAT), jnp.float32),
            pltpu.VMEM_SHARED((ACC_ROWS, D_FEAT), jnp.float32),
            pltpu.SemaphoreType.DMA,
        ],
    )


def _tc_add_body(parts_ref, out_ref):
    out_ref[...] = parts_ref[0] + parts_ref[1]


def _tc_add(parts):
    rows = 2000
    return pl.pallas_call(
        _tc_add_body,
        out_shape=jax.ShapeDtypeStruct((N_NODES, D_FEAT), jnp.float32),
        grid=(N_NODES // rows,),
        in_specs=[pl.BlockSpec((NC, rows, D_FEAT), lambda i: (0, i, 0))],
        out_specs=pl.BlockSpec((rows, D_FEAT), lambda i: (i, 0)),
    )(parts)


@jax.jit
def kernel(entity_emb, relation_emb, scores, edge_index, edge_type):
    head = edge_index[0].astype(jnp.int32)
    tail = edge_index[1].astype(jnp.int32)
    rel_idx = jnp.remainder(edge_type.astype(jnp.int32) - 1, N_REL)
    parts = _get_sc_agg()(entity_emb, relation_emb, scores, head, tail,
                          rel_idx)
    return _tc_add(parts)
